# baseline (speedup 1.0000x reference)
import functools

import jax
import jax.numpy as jnp
from jax import lax
from jax.experimental import pallas as pl
from jax.experimental.pallas import tpu as pltpu
from jax.experimental.pallas import tpu_sc as plsc

V = 64
H = 384
W = 384
HW = H * W

NC = 2
NS = 16
NW = NC * NS
VIEWS_PER_W = V // NW
ROWS_PER_CHUNK = 12
P = ROWS_PER_CHUNK * W
CH = P * 3
NCHUNK = HW // P
QSTEPS = W // 16

_mesh = plsc.VectorSubcoreMesh(
    core_axis_name="c", subcore_axis_name="s", num_cores=NC, num_subcores=NS)


@functools.partial(
    pl.kernel,
    out_type=jax.ShapeDtypeStruct((V * HW * 3,), jnp.float32),
    mesh=_mesh,
    scratch_types=[
        pltpu.VMEM((CH,), jnp.float32),
        pltpu.VMEM((CH,), jnp.float32),
        pltpu.VMEM((16,), jnp.float32),
    ],
    compiler_params=pltpu.CompilerParams(needs_layout_passes=False),
)
def _sc_transform(pts_hbm, coef_hbm, out_hbm, in_v, out_v, coef_v):
    cid = lax.axis_index("c")
    sid = lax.axis_index("s")
    wid = sid * NC + cid
    lanes = lax.iota(jnp.int32, 16)
    lanes3 = lanes * 3
    lanes_f = lanes.astype(jnp.float32)

    def view_body(vi, _):
        v = wid * VIEWS_PER_W + vi
        pltpu.sync_copy(coef_hbm.at[v], coef_v)
        row = coef_v[...]

        def bcast(j):
            return jnp.take(row, lanes * 0 + j)

        a0, a1, a2 = bcast(0), bcast(1), bcast(2)
        b0, b1, b2 = bcast(3), bcast(4), bcast(5)
        c0, c1, c2 = bcast(6), bcast(7), bcast(8)
        t0, t1, t2 = bcast(9), bcast(10), bcast(11)

        def chunk_body(ci, _):
            base = (v * HW + ci * P) * 3
            pltpu.sync_copy(pts_hbm.at[pl.ds(base, CH)], in_v)

            def row_body(r, _):
                py = (ci * ROWS_PER_CHUNK + r).astype(jnp.float32)
                e0 = b0 * py + c0
                e1 = b1 * py + c1
                e2 = b2 * py + c2
                row3 = r * (W * 3)

                def q_body(q, _):
                    px = q.astype(jnp.float32) * 16.0 + lanes_f
                    ib = row3 + q * 48 + lanes3
                    d = plsc.load_gather(in_v, [ib + 2])
                    x = d * (a0 * px + e0) + t0
                    y = d * (a1 * px + e1) + t1
                    z = d * (a2 * px + e2) + t2
                    plsc.store_scatter(out_v, [ib], x)
                    plsc.store_scatter(out_v, [ib + 1], y)
                    plsc.store_scatter(out_v, [ib + 2], z)
                    return 0

                return lax.fori_loop(0, QSTEPS, q_body, 0)

            lax.fori_loop(0, ROWS_PER_CHUNK, row_body, 0)
            pltpu.sync_copy(out_v, out_hbm.at[pl.ds(base, CH)])
            return 0

        return lax.fori_loop(0, NCHUNK, chunk_body, 0)

    lax.fori_loop(0, VIEWS_PER_W, view_body, 0)


def kernel(pts3d_cam, pixels, focals, pp, poses):
    del pixels
    fx = focals[:, 0:1]
    fy = focals[:, 1:2]
    a = poses[:, :3, 0] / fx
    b = poses[:, :3, 1] / fy
    c = poses[:, :3, 2] - a * pp[:, 0:1] - b * pp[:, 1:2]
    t = poses[:, :3, 3]
    coef = jnp.concatenate(
        [a, b, c, t, jnp.zeros((V, 4), jnp.float32)], axis=1)
    out_flat = _sc_transform(pts3d_cam.reshape(V * HW * 3), coef)
    return out_flat.reshape(V, HW, 3)


# speedup vs baseline: 107.9806x; 107.9806x over previous
import functools

import jax
import jax.numpy as jnp
from jax.experimental import pallas as pl
from jax.experimental.pallas import tpu as pltpu

V = 64
H = 384
W = 384
HW = H * W
B = 2048


def _body(pts_ref, pix_ref, coef_ref, out_ref):
    d = pts_ref[0]
    px = pix_ref[0:1, :]
    py = pix_ref[1:2, :]
    cf = coef_ref[...]
    for j in range(3):
        a = cf[:, j:j + 1]
        b = cf[:, 3 + j:4 + j]
        c = cf[:, 6 + j:7 + j]
        t = cf[:, 9 + j:10 + j]
        out_ref[j] = d * (a * px + b * py + c) + t


@jax.jit
def kernel(pts3d_cam, pixels, focals, pp, poses):
    ptsT = jnp.transpose(pts3d_cam, (2, 0, 1))
    pixT = jnp.transpose(pixels, (1, 0))
    fx = focals[:, 0:1]
    fy = focals[:, 1:2]
    a = poses[:, :3, 0] / fx
    b = poses[:, :3, 1] / fy
    c = poses[:, :3, 2] - a * pp[:, 0:1] - b * pp[:, 1:2]
    t = poses[:, :3, 3]
    coef = jnp.concatenate([a, b, c, t], axis=1)

    out = pl.pallas_call(
        _body,
        grid=(HW // B, V // 8),
        in_specs=[
            pl.BlockSpec((1, 8, B), lambda pb, vb: (2, vb, pb)),
            pl.BlockSpec((2, B), lambda pb, vb: (0, pb)),
            pl.BlockSpec((8, 12), lambda pb, vb: (vb, 0)),
        ],
        out_specs=pl.BlockSpec((3, 8, B), lambda pb, vb: (0, vb, pb)),
        out_shape=jax.ShapeDtypeStruct((3, V, HW), jnp.float32),
        compiler_params=pltpu.CompilerParams(
            dimension_semantics=("parallel", "parallel")),
    )(ptsT, pixT, coef)
    return jnp.transpose(out, (1, 2, 0))


# speedup vs baseline: 187.0125x; 1.7319x over previous
import functools

import jax
import jax.numpy as jnp
from jax.experimental import pallas as pl
from jax.experimental.pallas import tpu as pltpu

V = 64
H = 384
W = 384
HW = H * W
B = 4096


def _body(pts_ref, pix_ref, coef_ref, out_ref):
    d = pts_ref[0]
    px = pix_ref[0:1, :]
    py = pix_ref[1:2, :]
    cf = coef_ref[...]
    for j in range(3):
        a = cf[:, j:j + 1]
        b = cf[:, 3 + j:4 + j]
        c = cf[:, 6 + j:7 + j]
        t = cf[:, 9 + j:10 + j]
        out_ref[j] = d * (a * px + b * py + c) + t


@jax.jit
def kernel(pts3d_cam, pixels, focals, pp, poses):
    ptsT = jnp.transpose(pts3d_cam, (2, 0, 1))
    pixT = jnp.transpose(pixels, (1, 0))
    fx = focals[:, 0:1]
    fy = focals[:, 1:2]
    a = poses[:, :3, 0] / fx
    b = poses[:, :3, 1] / fy
    c = poses[:, :3, 2] - a * pp[:, 0:1] - b * pp[:, 1:2]
    t = poses[:, :3, 3]
    coef = jnp.concatenate([a, b, c, t], axis=1)

    out = pl.pallas_call(
        _body,
        grid=(HW // B, V // 8),
        in_specs=[
            pl.BlockSpec((1, 8, B), lambda pb, vb: (2, vb, pb)),
            pl.BlockSpec((2, B), lambda pb, vb: (0, pb)),
            pl.BlockSpec((8, 12), lambda pb, vb: (vb, 0)),
        ],
        out_specs=pl.BlockSpec((3, 8, B), lambda pb, vb: (0, vb, pb)),
        out_shape=jax.ShapeDtypeStruct((3, V, HW), jnp.float32),
        compiler_params=pltpu.CompilerParams(
            dimension_semantics=("parallel", "parallel")),
    )(ptsT, pixT, coef)
    return jnp.transpose(out, (1, 2, 0))


# speedup vs baseline: 290.6271x; 1.5541x over previous
import functools

import jax
import jax.numpy as jnp
from jax.experimental import pallas as pl
from jax.experimental.pallas import tpu as pltpu

V = 64
H = 384
W = 384
HW = H * W
B = 8192


def _body(pts_ref, pix_ref, coef_ref, out_ref):
    d = pts_ref[0]
    px = pix_ref[0:1, :]
    py = pix_ref[1:2, :]
    cf = coef_ref[...]
    for j in range(3):
        a = cf[:, j:j + 1]
        b = cf[:, 3 + j:4 + j]
        c = cf[:, 6 + j:7 + j]
        t = cf[:, 9 + j:10 + j]
        out_ref[j] = d * (a * px + b * py + c) + t


@jax.jit
def kernel(pts3d_cam, pixels, focals, pp, poses):
    ptsT = jnp.transpose(pts3d_cam, (2, 0, 1))
    pixT = jnp.transpose(pixels, (1, 0))
    fx = focals[:, 0:1]
    fy = focals[:, 1:2]
    a = poses[:, :3, 0] / fx
    b = poses[:, :3, 1] / fy
    c = poses[:, :3, 2] - a * pp[:, 0:1] - b * pp[:, 1:2]
    t = poses[:, :3, 3]
    coef = jnp.concatenate([a, b, c, t], axis=1)

    out = pl.pallas_call(
        _body,
        grid=(HW // B, V // 8),
        in_specs=[
            pl.BlockSpec((1, 8, B), lambda pb, vb: (2, vb, pb)),
            pl.BlockSpec((2, B), lambda pb, vb: (0, pb)),
            pl.BlockSpec((8, 12), lambda pb, vb: (vb, 0)),
        ],
        out_specs=pl.BlockSpec((3, 8, B), lambda pb, vb: (0, vb, pb)),
        out_shape=jax.ShapeDtypeStruct((3, V, HW), jnp.float32),
        compiler_params=pltpu.CompilerParams(
            dimension_semantics=("parallel", "parallel")),
    )(ptsT, pixT, coef)
    return jnp.transpose(out, (1, 2, 0))


# speedup vs baseline: 407.4719x; 1.4020x over previous
import functools

import jax
import jax.numpy as jnp
from jax.experimental import pallas as pl
from jax.experimental.pallas import tpu as pltpu

V = 64
H = 384
W = 384
HW = H * W
B = 16384


def _body(pts_ref, pix_ref, coef_ref, out_ref):
    d = pts_ref[0]
    px = pix_ref[0:1, :]
    py = pix_ref[1:2, :]
    cf = coef_ref[...]
    for j in range(3):
        a = cf[:, j:j + 1]
        b = cf[:, 3 + j:4 + j]
        c = cf[:, 6 + j:7 + j]
        t = cf[:, 9 + j:10 + j]
        out_ref[j] = d * (a * px + b * py + c) + t


@jax.jit
def kernel(pts3d_cam, pixels, focals, pp, poses):
    ptsT = jnp.transpose(pts3d_cam, (2, 0, 1))
    pixT = jnp.transpose(pixels, (1, 0))
    fx = focals[:, 0:1]
    fy = focals[:, 1:2]
    a = poses[:, :3, 0] / fx
    b = poses[:, :3, 1] / fy
    c = poses[:, :3, 2] - a * pp[:, 0:1] - b * pp[:, 1:2]
    t = poses[:, :3, 3]
    coef = jnp.concatenate([a, b, c, t], axis=1)

    out = pl.pallas_call(
        _body,
        grid=(HW // B, V // 8),
        in_specs=[
            pl.BlockSpec((1, 8, B), lambda pb, vb: (2, vb, pb)),
            pl.BlockSpec((2, B), lambda pb, vb: (0, pb)),
            pl.BlockSpec((8, 12), lambda pb, vb: (vb, 0)),
        ],
        out_specs=pl.BlockSpec((3, 8, B), lambda pb, vb: (0, vb, pb)),
        out_shape=jax.ShapeDtypeStruct((3, V, HW), jnp.float32),
        compiler_params=pltpu.CompilerParams(
            dimension_semantics=("parallel", "parallel")),
    )(ptsT, pixT, coef)
    return jnp.transpose(out, (1, 2, 0))


# speedup vs baseline: 464.9713x; 1.1411x over previous
import functools

import jax
import jax.numpy as jnp
from jax.experimental import pallas as pl
from jax.experimental.pallas import tpu as pltpu

V = 64
H = 384
W = 384
HW = H * W
B = 24576


def _body(pts_ref, pix_ref, coef_ref, out_ref):
    d = pts_ref[0]
    px = pix_ref[0:1, :]
    py = pix_ref[1:2, :]
    cf = coef_ref[...]
    for j in range(3):
        a = cf[:, j:j + 1]
        b = cf[:, 3 + j:4 + j]
        c = cf[:, 6 + j:7 + j]
        t = cf[:, 9 + j:10 + j]
        out_ref[j] = d * (a * px + b * py + c) + t


@jax.jit
def kernel(pts3d_cam, pixels, focals, pp, poses):
    ptsT = jnp.transpose(pts3d_cam, (2, 0, 1))
    pixT = jnp.transpose(pixels, (1, 0))
    fx = focals[:, 0:1]
    fy = focals[:, 1:2]
    a = poses[:, :3, 0] / fx
    b = poses[:, :3, 1] / fy
    c = poses[:, :3, 2] - a * pp[:, 0:1] - b * pp[:, 1:2]
    t = poses[:, :3, 3]
    coef = jnp.concatenate([a, b, c, t], axis=1)

    out = pl.pallas_call(
        _body,
        grid=(HW // B, V // 8),
        in_specs=[
            pl.BlockSpec((1, 8, B), lambda pb, vb: (2, vb, pb)),
            pl.BlockSpec((2, B), lambda pb, vb: (0, pb)),
            pl.BlockSpec((8, 12), lambda pb, vb: (vb, 0)),
        ],
        out_specs=pl.BlockSpec((3, 8, B), lambda pb, vb: (0, vb, pb)),
        out_shape=jax.ShapeDtypeStruct((3, V, HW), jnp.float32),
        compiler_params=pltpu.CompilerParams(
            dimension_semantics=("parallel", "parallel")),
    )(ptsT, pixT, coef)
    return jnp.transpose(out, (1, 2, 0))


# speedup vs baseline: 516.4610x; 1.1107x over previous
import functools

import jax
import jax.numpy as jnp
from jax.experimental import pallas as pl
from jax.experimental.pallas import tpu as pltpu

V = 64
H = 384
W = 384
HW = H * W
B = 36864


def _body(pts_ref, pix_ref, coef_ref, out_ref):
    d = pts_ref[0]
    px = pix_ref[0:1, :]
    py = pix_ref[1:2, :]
    cf = coef_ref[...]
    for j in range(3):
        a = cf[:, j:j + 1]
        b = cf[:, 3 + j:4 + j]
        c = cf[:, 6 + j:7 + j]
        t = cf[:, 9 + j:10 + j]
        out_ref[j] = d * (a * px + b * py + c) + t


@jax.jit
def kernel(pts3d_cam, pixels, focals, pp, poses):
    ptsT = jnp.transpose(pts3d_cam, (2, 0, 1))
    pixT = jnp.transpose(pixels, (1, 0))
    fx = focals[:, 0:1]
    fy = focals[:, 1:2]
    a = poses[:, :3, 0] / fx
    b = poses[:, :3, 1] / fy
    c = poses[:, :3, 2] - a * pp[:, 0:1] - b * pp[:, 1:2]
    t = poses[:, :3, 3]
    coef = jnp.concatenate([a, b, c, t], axis=1)

    out = pl.pallas_call(
        _body,
        grid=(HW // B, V // 8),
        in_specs=[
            pl.BlockSpec((1, 8, B), lambda pb, vb: (2, vb, pb)),
            pl.BlockSpec((2, B), lambda pb, vb: (0, pb)),
            pl.BlockSpec((8, 12), lambda pb, vb: (vb, 0)),
        ],
        out_specs=pl.BlockSpec((3, 8, B), lambda pb, vb: (0, vb, pb)),
        out_shape=jax.ShapeDtypeStruct((3, V, HW), jnp.float32),
        compiler_params=pltpu.CompilerParams(
            dimension_semantics=("parallel", "parallel")),
    )(ptsT, pixT, coef)
    return jnp.transpose(out, (1, 2, 0))


# speedup vs baseline: 547.2417x; 1.0596x over previous
import functools

import jax
import jax.numpy as jnp
from jax.experimental import pallas as pl
from jax.experimental.pallas import tpu as pltpu

V = 64
H = 384
W = 384
HW = H * W
B = 49152


def _body(pts_ref, pix_ref, coef_ref, out_ref):
    d = pts_ref[0]
    px = pix_ref[0:1, :]
    py = pix_ref[1:2, :]
    cf = coef_ref[...]
    for j in range(3):
        a = cf[:, j:j + 1]
        b = cf[:, 3 + j:4 + j]
        c = cf[:, 6 + j:7 + j]
        t = cf[:, 9 + j:10 + j]
        out_ref[j] = d * (a * px + b * py + c) + t


@jax.jit
def kernel(pts3d_cam, pixels, focals, pp, poses):
    ptsT = jnp.transpose(pts3d_cam, (2, 0, 1))
    pixT = jnp.transpose(pixels, (1, 0))
    fx = focals[:, 0:1]
    fy = focals[:, 1:2]
    a = poses[:, :3, 0] / fx
    b = poses[:, :3, 1] / fy
    c = poses[:, :3, 2] - a * pp[:, 0:1] - b * pp[:, 1:2]
    t = poses[:, :3, 3]
    coef = jnp.concatenate([a, b, c, t], axis=1)

    out = pl.pallas_call(
        _body,
        grid=(HW // B, V // 8),
        in_specs=[
            pl.BlockSpec((1, 8, B), lambda pb, vb: (2, vb, pb)),
            pl.BlockSpec((2, B), lambda pb, vb: (0, pb)),
            pl.BlockSpec((8, 12), lambda pb, vb: (vb, 0)),
        ],
        out_specs=pl.BlockSpec((3, 8, B), lambda pb, vb: (0, vb, pb)),
        out_shape=jax.ShapeDtypeStruct((3, V, HW), jnp.float32),
        compiler_params=pltpu.CompilerParams(
            dimension_semantics=("parallel", "parallel")),
    )(ptsT, pixT, coef)
    return jnp.transpose(out, (1, 2, 0))


# speedup vs baseline: 579.1500x; 1.0583x over previous
import functools

import jax
import jax.numpy as jnp
from jax.experimental import pallas as pl
from jax.experimental.pallas import tpu as pltpu

V = 64
H = 384
W = 384
HW = H * W
B = 73728


def _body(pts_ref, pix_ref, coef_ref, out_ref):
    d = pts_ref[0]
    px = pix_ref[0:1, :]
    py = pix_ref[1:2, :]
    cf = coef_ref[...]
    for j in range(3):
        a = cf[:, j:j + 1]
        b = cf[:, 3 + j:4 + j]
        c = cf[:, 6 + j:7 + j]
        t = cf[:, 9 + j:10 + j]
        out_ref[j] = d * (a * px + b * py + c) + t


@jax.jit
def kernel(pts3d_cam, pixels, focals, pp, poses):
    ptsT = jnp.transpose(pts3d_cam, (2, 0, 1))
    pixT = jnp.transpose(pixels, (1, 0))
    fx = focals[:, 0:1]
    fy = focals[:, 1:2]
    a = poses[:, :3, 0] / fx
    b = poses[:, :3, 1] / fy
    c = poses[:, :3, 2] - a * pp[:, 0:1] - b * pp[:, 1:2]
    t = poses[:, :3, 3]
    coef = jnp.concatenate([a, b, c, t], axis=1)

    out = pl.pallas_call(
        _body,
        grid=(HW // B, V // 8),
        in_specs=[
            pl.BlockSpec((1, 8, B), lambda pb, vb: (2, vb, pb)),
            pl.BlockSpec((2, B), lambda pb, vb: (0, pb)),
            pl.BlockSpec((8, 12), lambda pb, vb: (vb, 0)),
        ],
        out_specs=pl.BlockSpec((3, 8, B), lambda pb, vb: (0, vb, pb)),
        out_shape=jax.ShapeDtypeStruct((3, V, HW), jnp.float32),
        compiler_params=pltpu.CompilerParams(
            dimension_semantics=("parallel", "parallel")),
    )(ptsT, pixT, coef)
    return jnp.transpose(out, (1, 2, 0))
